# Initial kernel scaffold; baseline (speedup 1.0000x reference)
import jax
import jax.numpy as jnp
from jax.experimental import pallas as pl


def kernel(x, edge_index, batch, W1, b1, W2, b2, W3, b3):
    raise NotImplementedError("write your pallas kernel here")


# speedup vs baseline: 9.1310x; 9.1310x over previous
import functools

import jax
import jax.numpy as jnp
from jax import lax
from jax.experimental import pallas as pl
from jax.experimental.pallas import tpu as pltpu
from jax.experimental.pallas import tpu_sc as plsc

N = 10000
E = 320000
G = 64
D = 128

NC = 2
NS = 16
NW = NC * NS

N_PAD = 10240
ROWS_PER_TILE = N_PAD // NS
EK = 128
EPT = 10112
E_PAD = EPT * NW
N_CHUNKS = EPT // EK

@functools.cache
def _sc_mesh():
    return plsc.VectorSubcoreMesh(
        core_axis_name="c", subcore_axis_name="s", num_cores=NC, num_subcores=NS)


def _deg_body(col_hbm, ones_hbm, zeros_hbm, out_hbm, cbuf, onesbuf, acc):
    cid = lax.axis_index("c")
    sid = lax.axis_index("s")
    wid = sid * NC + cid

    pltpu.sync_copy(ones_hbm, onesbuf)
    row0 = pl.multiple_of(sid * ROWS_PER_TILE, 8)
    pltpu.sync_copy(zeros_hbm, acc.at[pl.ds(row0, ROWS_PER_TILE)])
    plsc.subcore_barrier()

    ebase = wid * EPT

    def body(i, carry):
        off = pl.multiple_of(ebase + i * EK, 8)
        pltpu.sync_copy(col_hbm.at[pl.ds(off, EK)], cbuf)
        pltpu.sync_copy(onesbuf, acc.at[cbuf], add=True)
        return carry

    lax.fori_loop(0, N_CHUNKS, body, 0)
    plsc.subcore_barrier()

    out0 = pl.multiple_of(cid * N_PAD + sid * ROWS_PER_TILE, 8)
    pltpu.sync_copy(acc.at[pl.ds(row0, ROWS_PER_TILE)],
                    out_hbm.at[pl.ds(out0, ROWS_PER_TILE)])


@functools.cache
def _deg_call():
    return pl.kernel(
        _deg_body,
        out_type=jax.ShapeDtypeStruct((NC * N_PAD,), jnp.float32),
        mesh=_sc_mesh(),
        scratch_types=[
            pltpu.VMEM((EK,), jnp.int32),
            pltpu.VMEM((EK,), jnp.float32),
            pltpu.VMEM_SHARED((N_PAD,), jnp.float32),
        ],
    )


def _agg_body(row_hbm, col_hbm, u_hbm, zeros_hbm, out_hbm,
              rbuf, cbuf, gbuf, acc, sem):
    cid = lax.axis_index("c")
    sid = lax.axis_index("s")
    wid = sid * NC + cid

    row0 = pl.multiple_of(sid * ROWS_PER_TILE, 8)
    pltpu.sync_copy(zeros_hbm, acc.at[pl.ds(row0, ROWS_PER_TILE)])
    plsc.subcore_barrier()

    ebase = wid * EPT

    def body(i, carry):
        off = pl.multiple_of(ebase + i * EK, 8)
        pltpu.sync_copy(row_hbm.at[pl.ds(off, EK)], rbuf)
        pltpu.sync_copy(col_hbm.at[pl.ds(off, EK)], cbuf)
        pltpu.async_copy(u_hbm.at[rbuf], gbuf, sem).wait()
        pltpu.sync_copy(gbuf, acc.at[cbuf], add=True)
        return carry

    lax.fori_loop(0, N_CHUNKS, body, 0)
    plsc.subcore_barrier()

    out0 = pl.multiple_of(cid * N_PAD + sid * ROWS_PER_TILE, 8)
    pltpu.sync_copy(acc.at[pl.ds(row0, ROWS_PER_TILE)],
                    out_hbm.at[pl.ds(out0, ROWS_PER_TILE)])


@functools.cache
def _agg_call():
    return pl.kernel(
        _agg_body,
        out_type=jax.ShapeDtypeStruct((NC * N_PAD, D), jnp.float32),
        mesh=_sc_mesh(),
        scratch_types=[
            pltpu.VMEM((EK,), jnp.int32),
            pltpu.VMEM((EK,), jnp.int32),
            pltpu.VMEM((EK, D), jnp.float32),
            pltpu.VMEM_SHARED((N_PAD, D), jnp.float32),
            pltpu.SemaphoreType.DMA,
        ],
    )


_BLK = 1000


def _layer1_body(x_ref, w_ref, d0_ref, d1_ref, u_ref, dinv_ref):
    dinv = lax.rsqrt(d0_ref[...] + d1_ref[...] + 1.0)
    dinv_ref[...] = dinv
    h = jnp.dot(x_ref[...], w_ref[...], preferred_element_type=jnp.float32)
    u_ref[...] = dinv * h


def _layer1(x, W, deg0, deg1):
    return pl.pallas_call(
        _layer1_body,
        grid=(N // _BLK,),
        in_specs=[
            pl.BlockSpec((_BLK, D), lambda i: (i, 0)),
            pl.BlockSpec((D, D), lambda i: (0, 0)),
            pl.BlockSpec((_BLK, 1), lambda i: (i, 0)),
            pl.BlockSpec((_BLK, 1), lambda i: (i, 0)),
        ],
        out_specs=[
            pl.BlockSpec((_BLK, D), lambda i: (i, 0)),
            pl.BlockSpec((_BLK, 1), lambda i: (i, 0)),
        ],
        out_shape=[
            jax.ShapeDtypeStruct((N, D), jnp.float32),
            jax.ShapeDtypeStruct((N, 1), jnp.float32),
        ],
    )(x, W, deg0, deg1)


def _mid_body(s0_ref, s1_ref, up_ref, dinv_ref, b_ref, w_ref, u_ref):
    dinv = dinv_ref[...]
    s = s0_ref[...] + s1_ref[...] + up_ref[...]
    y = jax.nn.relu(dinv * s + b_ref[...])
    u_ref[...] = dinv * jnp.dot(y, w_ref[...], preferred_element_type=jnp.float32)


def _mid_layer(s0, s1, u_prev, dinv, b, W):
    return pl.pallas_call(
        _mid_body,
        grid=(N // _BLK,),
        in_specs=[
            pl.BlockSpec((_BLK, D), lambda i: (i, 0)),
            pl.BlockSpec((_BLK, D), lambda i: (i, 0)),
            pl.BlockSpec((_BLK, D), lambda i: (i, 0)),
            pl.BlockSpec((_BLK, 1), lambda i: (i, 0)),
            pl.BlockSpec((1, D), lambda i: (0, 0)),
            pl.BlockSpec((D, D), lambda i: (0, 0)),
        ],
        out_specs=pl.BlockSpec((_BLK, D), lambda i: (i, 0)),
        out_shape=jax.ShapeDtypeStruct((N, D), jnp.float32),
    )(s0, s1, u_prev, dinv, b, W)


def _pool_body(s0_ref, s1_ref, up_ref, dinv_ref, b_ref, batch_ref,
               mean_ref, max_ref):
    s = s0_ref[...] + s1_ref[...] + up_ref[...]
    y = jax.nn.relu(dinv_ref[...] * s + b_ref[...])
    batch_row = batch_ref[...]
    gids = lax.broadcasted_iota(jnp.int32, (G, N), 0)
    mask = (batch_row == gids).astype(jnp.float32)
    seg_sum = jnp.dot(mask, y, preferred_element_type=jnp.float32)
    counts = jnp.sum(mask, axis=1, keepdims=True)
    mean_ref[...] = seg_sum / jnp.maximum(counts, 1.0)

    batch_col = batch_ref[...].reshape(N, 1)
    neg_inf = jnp.float32(-jnp.inf)

    def body(g, carry):
        sel = jnp.where(batch_col == g, y, neg_inf)
        max_ref[pl.ds(g, 1), :] = jnp.max(sel, axis=0, keepdims=True)
        return carry

    lax.fori_loop(0, G, body, 0)


def _pool(s0, s1, u_prev, dinv, b, batch2d):
    return pl.pallas_call(
        _pool_body,
        out_shape=[
            jax.ShapeDtypeStruct((G, D), jnp.float32),
            jax.ShapeDtypeStruct((G, D), jnp.float32),
        ],
    )(s0, s1, u_prev, dinv, b, batch2d)


def kernel(x, edge_index, batch, W1, b1, W2, b2, W3, b3):
    row = edge_index[0].astype(jnp.int32)
    col = edge_index[1].astype(jnp.int32)
    pad = E_PAD - E
    row_pad = jnp.concatenate([row, jnp.zeros((pad,), jnp.int32)])
    col_pad = jnp.concatenate([col, jnp.full((pad,), N, jnp.int32)])

    ones_ek = jnp.ones((EK,), jnp.float32)
    zeros_1d = jnp.zeros((ROWS_PER_TILE,), jnp.float32)
    zeros_2d = jnp.zeros((ROWS_PER_TILE, D), jnp.float32)

    deg_parts = _deg_call()(col_pad, ones_ek, zeros_1d)
    deg0 = deg_parts[:N].reshape(N, 1)
    deg1 = deg_parts[N_PAD:N_PAD + N].reshape(N, 1)

    def agg(u):
        s_parts = _agg_call()(row_pad, col_pad, u, zeros_2d)
        return s_parts[:N], s_parts[N_PAD:N_PAD + N]

    u1, dinv = _layer1(x, W1, deg0, deg1)
    s0, s1 = agg(u1)
    u2 = _mid_layer(s0, s1, u1, dinv, b1.reshape(1, D), W2)
    s0, s1 = agg(u2)
    u3 = _mid_layer(s0, s1, u2, dinv, b2.reshape(1, D), W3)
    s0, s1 = agg(u3)
    mean, mx = _pool(s0, s1, u3, dinv, b3.reshape(1, D), batch.reshape(1, N))
    return jnp.concatenate([mean, mx], axis=-1)


# speedup vs baseline: 11.4853x; 1.2578x over previous
import functools

import jax
import jax.numpy as jnp
from jax import lax
from jax.experimental import pallas as pl
from jax.experimental.pallas import tpu as pltpu
from jax.experimental.pallas import tpu_sc as plsc

N = 10000
E = 320000
G = 64
D = 128

NC = 2
NS = 16
NW = NC * NS

N_PAD = 10240
ROWS_PER_TILE = N_PAD // NS
EK = 128
EPT = 10112
E_PAD = EPT * NW
N_CHUNKS = EPT // EK

GK = 64
G_CHUNKS = EPT // GK
NBUF = 2

@functools.cache
def _sc_mesh():
    return plsc.VectorSubcoreMesh(
        core_axis_name="c", subcore_axis_name="s", num_cores=NC, num_subcores=NS)


def _deg_body(col_hbm, ones_hbm, zeros_hbm, out_hbm, cbuf, onesbuf, acc):
    cid = lax.axis_index("c")
    sid = lax.axis_index("s")
    wid = sid * NC + cid

    pltpu.sync_copy(ones_hbm, onesbuf)
    row0 = pl.multiple_of(sid * ROWS_PER_TILE, 8)
    pltpu.sync_copy(zeros_hbm, acc.at[pl.ds(row0, ROWS_PER_TILE)])
    plsc.subcore_barrier()

    ebase = wid * EPT

    def body(i, carry):
        off = pl.multiple_of(ebase + i * EK, 8)
        pltpu.sync_copy(col_hbm.at[pl.ds(off, EK)], cbuf)
        pltpu.sync_copy(onesbuf, acc.at[cbuf], add=True)
        return carry

    lax.fori_loop(0, N_CHUNKS, body, 0)
    plsc.subcore_barrier()

    out0 = pl.multiple_of(cid * N_PAD + sid * ROWS_PER_TILE, 8)
    pltpu.sync_copy(acc.at[pl.ds(row0, ROWS_PER_TILE)],
                    out_hbm.at[pl.ds(out0, ROWS_PER_TILE)])


@functools.cache
def _deg_call():
    return pl.kernel(
        _deg_body,
        out_type=jax.ShapeDtypeStruct((NC * N_PAD,), jnp.float32),
        mesh=_sc_mesh(),
        scratch_types=[
            pltpu.VMEM((EK,), jnp.int32),
            pltpu.VMEM((EK,), jnp.float32),
            pltpu.VMEM_SHARED((N_PAD,), jnp.float32),
        ],
    )


def _agg_body(packed_hbm, u_hbm, zeros_hbm, out_hbm,
              pbuf, r0buf, r1buf, c0buf, c1buf, gbuf0, gbuf1, acc, sem):
    cid = lax.axis_index("c")
    sid = lax.axis_index("s")
    wid = sid * NC + cid
    gbufs = [gbuf0, gbuf1]
    rbufs = [r0buf, r1buf]
    cbufs = [c0buf, c1buf]

    pltpu.sync_copy(packed_hbm.at[wid], pbuf)

    row0 = pl.multiple_of(sid * ROWS_PER_TILE, 8)
    pltpu.sync_copy(zeros_hbm, acc.at[pl.ds(row0, ROWS_PER_TILE)])

    def unpack(i, b):
        for j in range(GK // 16):
            v = pbuf[i, pl.ds(j * 16, 16)]
            rbufs[b][pl.ds(j * 16, 16)] = lax.bitwise_and(v, 0xFFFF)
            cbufs[b][pl.ds(j * 16, 16)] = lax.shift_right_logical(v, 16)

    plsc.subcore_barrier()

    for b in range(NBUF):
        unpack(b, b)
        pltpu.async_copy(u_hbm.at[rbufs[b]], gbufs[b], sem)

    def body(i0, carry):
        for b in range(NBUF):
            i = i0 * NBUF + b
            pltpu.make_async_copy(u_hbm.at[rbufs[b]], gbufs[b], sem).wait()
            pltpu.sync_copy(gbufs[b], acc.at[cbufs[b]], add=True)
            unpack(i + NBUF, b)
            pltpu.async_copy(u_hbm.at[rbufs[b]], gbufs[b], sem)
        return carry

    lax.fori_loop(0, G_CHUNKS // NBUF - 1, body, 0)
    for b in range(NBUF):
        pltpu.make_async_copy(u_hbm.at[rbufs[b]], gbufs[b], sem).wait()
        pltpu.sync_copy(gbufs[b], acc.at[cbufs[b]], add=True)
    plsc.subcore_barrier()

    out0 = pl.multiple_of(cid * N_PAD + sid * ROWS_PER_TILE, 8)
    pltpu.sync_copy(acc.at[pl.ds(row0, ROWS_PER_TILE)],
                    out_hbm.at[pl.ds(out0, ROWS_PER_TILE)])


@functools.cache
def _agg_call():
    return pl.kernel(
        _agg_body,
        out_type=jax.ShapeDtypeStruct((NC * N_PAD, D), jnp.float32),
        mesh=_sc_mesh(),
        scratch_types=[
            pltpu.VMEM((G_CHUNKS, GK), jnp.int32),
            pltpu.VMEM((GK,), jnp.int32),
            pltpu.VMEM((GK,), jnp.int32),
            pltpu.VMEM((GK,), jnp.int32),
            pltpu.VMEM((GK,), jnp.int32),
            pltpu.VMEM((GK, D), jnp.float32),
            pltpu.VMEM((GK, D), jnp.float32),
            pltpu.VMEM_SHARED((N_PAD, D), jnp.float32),
            pltpu.SemaphoreType.DMA,
        ],
    )


_BLK = 1000


def _layer1_body(x_ref, w_ref, d0_ref, d1_ref, u_ref, dinv_ref):
    dinv = lax.rsqrt(d0_ref[...] + d1_ref[...] + 1.0)
    dinv_ref[...] = dinv
    h = jnp.dot(x_ref[...], w_ref[...], preferred_element_type=jnp.float32)
    u_ref[...] = dinv * h


def _layer1(x, W, deg0, deg1):
    return pl.pallas_call(
        _layer1_body,
        grid=(N // _BLK,),
        in_specs=[
            pl.BlockSpec((_BLK, D), lambda i: (i, 0)),
            pl.BlockSpec((D, D), lambda i: (0, 0)),
            pl.BlockSpec((_BLK, 1), lambda i: (i, 0)),
            pl.BlockSpec((_BLK, 1), lambda i: (i, 0)),
        ],
        out_specs=[
            pl.BlockSpec((_BLK, D), lambda i: (i, 0)),
            pl.BlockSpec((_BLK, 1), lambda i: (i, 0)),
        ],
        out_shape=[
            jax.ShapeDtypeStruct((N, D), jnp.float32),
            jax.ShapeDtypeStruct((N, 1), jnp.float32),
        ],
    )(x, W, deg0, deg1)


def _mid_body(s0_ref, s1_ref, up_ref, dinv_ref, b_ref, w_ref, u_ref):
    dinv = dinv_ref[...]
    s = s0_ref[...] + s1_ref[...] + up_ref[...]
    y = jax.nn.relu(dinv * s + b_ref[...])
    u_ref[...] = dinv * jnp.dot(y, w_ref[...], preferred_element_type=jnp.float32)


def _mid_layer(s0, s1, u_prev, dinv, b, W):
    return pl.pallas_call(
        _mid_body,
        grid=(N // _BLK,),
        in_specs=[
            pl.BlockSpec((_BLK, D), lambda i: (i, 0)),
            pl.BlockSpec((_BLK, D), lambda i: (i, 0)),
            pl.BlockSpec((_BLK, D), lambda i: (i, 0)),
            pl.BlockSpec((_BLK, 1), lambda i: (i, 0)),
            pl.BlockSpec((1, D), lambda i: (0, 0)),
            pl.BlockSpec((D, D), lambda i: (0, 0)),
        ],
        out_specs=pl.BlockSpec((_BLK, D), lambda i: (i, 0)),
        out_shape=jax.ShapeDtypeStruct((N, D), jnp.float32),
    )(s0, s1, u_prev, dinv, b, W)


def _pool_body(s0_ref, s1_ref, up_ref, dinv_ref, b_ref, batch_ref,
               mean_ref, max_ref):
    s = s0_ref[...] + s1_ref[...] + up_ref[...]
    y = jax.nn.relu(dinv_ref[...] * s + b_ref[...])
    batch_row = batch_ref[...]
    gids = lax.broadcasted_iota(jnp.int32, (G, N), 0)
    mask = (batch_row == gids).astype(jnp.float32)
    seg_sum = jnp.dot(mask, y, preferred_element_type=jnp.float32)
    counts = jnp.sum(mask, axis=1, keepdims=True)
    mean_ref[...] = seg_sum / jnp.maximum(counts, 1.0)

    batch_col = batch_ref[...].reshape(N, 1)
    neg_inf = jnp.float32(-jnp.inf)

    def body(g, carry):
        sel = jnp.where(batch_col == g, y, neg_inf)
        max_ref[pl.ds(g, 1), :] = jnp.max(sel, axis=0, keepdims=True)
        return carry

    lax.fori_loop(0, G, body, 0)


def _pool(s0, s1, u_prev, dinv, b, batch2d):
    return pl.pallas_call(
        _pool_body,
        out_shape=[
            jax.ShapeDtypeStruct((G, D), jnp.float32),
            jax.ShapeDtypeStruct((G, D), jnp.float32),
        ],
    )(s0, s1, u_prev, dinv, b, batch2d)


def kernel(x, edge_index, batch, W1, b1, W2, b2, W3, b3):
    row = edge_index[0].astype(jnp.int32)
    col = edge_index[1].astype(jnp.int32)
    pad = E_PAD - E
    row_pad = jnp.concatenate([row, jnp.zeros((pad,), jnp.int32)])
    col_pad = jnp.concatenate([col, jnp.full((pad,), N, jnp.int32)])

    ones_ek = jnp.ones((EK,), jnp.float32)
    zeros_1d = jnp.zeros((ROWS_PER_TILE,), jnp.float32)
    zeros_2d = jnp.zeros((ROWS_PER_TILE, D), jnp.float32)

    deg_parts = _deg_call()(col_pad, ones_ek, zeros_1d)
    deg0 = deg_parts[:N].reshape(N, 1)
    deg1 = deg_parts[N_PAD:N_PAD + N].reshape(N, 1)

    packed_3d = (row_pad | (col_pad << 16)).reshape(NW, G_CHUNKS, GK)

    def agg(u):
        s_parts = _agg_call()(packed_3d, u, zeros_2d)
        return s_parts[:N], s_parts[N_PAD:N_PAD + N]

    u1, dinv = _layer1(x, W1, deg0, deg1)
    s0, s1 = agg(u1)
    u2 = _mid_layer(s0, s1, u1, dinv, b1.reshape(1, D), W2)
    s0, s1 = agg(u2)
    u3 = _mid_layer(s0, s1, u2, dinv, b2.reshape(1, D), W3)
    s0, s1 = agg(u3)
    mean, mx = _pool(s0, s1, u3, dinv, b3.reshape(1, D), batch.reshape(1, N))
    return jnp.concatenate([mean, mx], axis=-1)


# speedup vs baseline: 19.2916x; 1.6797x over previous
import functools

import jax
import jax.numpy as jnp
from jax import lax
from jax.experimental import pallas as pl
from jax.experimental.pallas import tpu as pltpu
from jax.experimental.pallas import tpu_sc as plsc

N = 10000
E = 320000
G = 64
D = 128

NC = 2
NS = 16
NW = NC * NS

N_PAD = 10240
ROWS_PER_TILE = N_PAD // NS
EK = 128
EPT = 10112
E_PAD = EPT * NW
N_CHUNKS = EPT // EK

GK = 64
G_CHUNKS = EPT // GK
NBUF = 2

@functools.cache
def _sc_mesh():
    return plsc.VectorSubcoreMesh(
        core_axis_name="c", subcore_axis_name="s", num_cores=NC, num_subcores=NS)


def _deg_body(col_hbm, ones_hbm, zeros_hbm, out_hbm, cbuf, onesbuf, acc):
    cid = lax.axis_index("c")
    sid = lax.axis_index("s")
    wid = sid * NC + cid

    pltpu.sync_copy(ones_hbm, onesbuf)
    row0 = pl.multiple_of(sid * ROWS_PER_TILE, 8)
    pltpu.sync_copy(zeros_hbm, acc.at[pl.ds(row0, ROWS_PER_TILE)])
    plsc.subcore_barrier()

    ebase = wid * EPT

    def body(i, carry):
        off = pl.multiple_of(ebase + i * EK, 8)
        pltpu.sync_copy(col_hbm.at[pl.ds(off, EK)], cbuf)
        pltpu.sync_copy(onesbuf, acc.at[cbuf], add=True)
        return carry

    lax.fori_loop(0, N_CHUNKS, body, 0)
    plsc.subcore_barrier()

    out0 = pl.multiple_of(cid * N_PAD + sid * ROWS_PER_TILE, 8)
    pltpu.sync_copy(acc.at[pl.ds(row0, ROWS_PER_TILE)],
                    out_hbm.at[pl.ds(out0, ROWS_PER_TILE)])


@functools.cache
def _deg_call():
    return pl.kernel(
        _deg_body,
        out_type=jax.ShapeDtypeStruct((NC * N_PAD,), jnp.float32),
        mesh=_sc_mesh(),
        scratch_types=[
            pltpu.VMEM((EK,), jnp.int32),
            pltpu.VMEM((EK,), jnp.float32),
            pltpu.VMEM_SHARED((N_PAD,), jnp.float32),
        ],
    )


def _agg_body(packed_hbm, u_hbm, zeros_hbm, out_hbm,
              pbuf, r0buf, r1buf, c0buf, c1buf, gbuf0, gbuf1, acc, sem):
    cid = lax.axis_index("c")
    sid = lax.axis_index("s")
    wid = sid * NC + cid
    gbufs = [gbuf0, gbuf1]
    rbufs = [r0buf, r1buf]
    cbufs = [c0buf, c1buf]

    pltpu.sync_copy(packed_hbm.at[wid], pbuf)

    row0 = pl.multiple_of(sid * ROWS_PER_TILE, 8)
    pltpu.sync_copy(zeros_hbm, acc.at[pl.ds(row0, ROWS_PER_TILE)])

    def unpack(i, b):
        for j in range(GK // 16):
            v = pbuf[i, pl.ds(j * 16, 16)]
            rbufs[b][pl.ds(j * 16, 16)] = lax.bitwise_and(v, 0xFFFF)
            cbufs[b][pl.ds(j * 16, 16)] = lax.shift_right_logical(v, 16)

    plsc.subcore_barrier()

    for b in range(NBUF):
        unpack(b, b)
        pltpu.async_copy(u_hbm.at[rbufs[b]], gbufs[b], sem)

    def body(i0, carry):
        for b in range(NBUF):
            i = i0 * NBUF + b
            pltpu.make_async_copy(u_hbm.at[rbufs[b]], gbufs[b], sem).wait()
            pltpu.sync_copy(gbufs[b], acc.at[cbufs[b]], add=True)
            unpack(i + NBUF, b)
            pltpu.async_copy(u_hbm.at[rbufs[b]], gbufs[b], sem)
        return carry

    lax.fori_loop(0, G_CHUNKS // NBUF - 1, body, 0)
    for b in range(NBUF):
        pltpu.make_async_copy(u_hbm.at[rbufs[b]], gbufs[b], sem).wait()
        pltpu.sync_copy(gbufs[b], acc.at[cbufs[b]], add=True)
    plsc.subcore_barrier()

    out0 = pl.multiple_of(cid * N_PAD + sid * ROWS_PER_TILE, 8)
    pltpu.sync_copy(acc.at[pl.ds(row0, ROWS_PER_TILE)],
                    out_hbm.at[pl.ds(out0, ROWS_PER_TILE)])


@functools.cache
def _agg_call():
    return pl.kernel(
        _agg_body,
        out_type=jax.ShapeDtypeStruct((NC * N_PAD, D), jnp.float32),
        mesh=_sc_mesh(),
        scratch_types=[
            pltpu.VMEM((G_CHUNKS, GK), jnp.int32),
            pltpu.VMEM((GK,), jnp.int32),
            pltpu.VMEM((GK,), jnp.int32),
            pltpu.VMEM((GK,), jnp.int32),
            pltpu.VMEM((GK,), jnp.int32),
            pltpu.VMEM((GK, D), jnp.float32),
            pltpu.VMEM((GK, D), jnp.float32),
            pltpu.VMEM_SHARED((N_PAD, D), jnp.float32),
            pltpu.SemaphoreType.DMA,
        ],
    )


_BLK = 1000


def _layer1_body(x_ref, w_ref, d0_ref, d1_ref, u_ref, dinv_ref):
    dinv = lax.rsqrt(d0_ref[...] + d1_ref[...] + 1.0)
    dinv_ref[...] = dinv
    h = jnp.dot(x_ref[...], w_ref[...], preferred_element_type=jnp.float32)
    u_ref[...] = dinv * h


def _layer1(x, W, deg0, deg1):
    return pl.pallas_call(
        _layer1_body,
        grid=(N // _BLK,),
        in_specs=[
            pl.BlockSpec((_BLK, D), lambda i: (i, 0)),
            pl.BlockSpec((D, D), lambda i: (0, 0)),
            pl.BlockSpec((_BLK, 1), lambda i: (i, 0)),
            pl.BlockSpec((_BLK, 1), lambda i: (i, 0)),
        ],
        out_specs=[
            pl.BlockSpec((_BLK, D), lambda i: (i, 0)),
            pl.BlockSpec((_BLK, 1), lambda i: (i, 0)),
        ],
        out_shape=[
            jax.ShapeDtypeStruct((N, D), jnp.float32),
            jax.ShapeDtypeStruct((N, 1), jnp.float32),
        ],
    )(x, W, deg0, deg1)


def _mid_body(s0_ref, s1_ref, up_ref, dinv_ref, b_ref, w_ref, u_ref):
    dinv = dinv_ref[...]
    s = s0_ref[...] + s1_ref[...] + up_ref[...]
    y = jax.nn.relu(dinv * s + b_ref[...])
    u_ref[...] = dinv * jnp.dot(y, w_ref[...], preferred_element_type=jnp.float32)


def _mid_layer(s0, s1, u_prev, dinv, b, W):
    return pl.pallas_call(
        _mid_body,
        grid=(N // _BLK,),
        in_specs=[
            pl.BlockSpec((_BLK, D), lambda i: (i, 0)),
            pl.BlockSpec((_BLK, D), lambda i: (i, 0)),
            pl.BlockSpec((_BLK, D), lambda i: (i, 0)),
            pl.BlockSpec((_BLK, 1), lambda i: (i, 0)),
            pl.BlockSpec((1, D), lambda i: (0, 0)),
            pl.BlockSpec((D, D), lambda i: (0, 0)),
        ],
        out_specs=pl.BlockSpec((_BLK, D), lambda i: (i, 0)),
        out_shape=jax.ShapeDtypeStruct((N, D), jnp.float32),
    )(s0, s1, u_prev, dinv, b, W)


def _pool_body(s0_ref, s1_ref, up_ref, dinv_ref, b_ref, batch_ref,
               mean_ref, max_ref):
    s = s0_ref[...] + s1_ref[...] + up_ref[...]
    y = jax.nn.relu(dinv_ref[...] * s + b_ref[...])
    batch_row = batch_ref[...]
    gids = lax.broadcasted_iota(jnp.int32, (G, N), 0)
    mask = (batch_row == gids).astype(jnp.float32)
    seg_sum = jnp.dot(mask, y, preferred_element_type=jnp.float32)
    counts = jnp.sum(mask, axis=1, keepdims=True)
    mean_ref[...] = seg_sum / jnp.maximum(counts, 1.0)

    batch_col = batch_ref[...].reshape(N, 1)
    neg_inf = jnp.float32(-jnp.inf)

    def body(g, carry):
        sel = jnp.where(batch_col == g, y, neg_inf)
        max_ref[pl.ds(g, 1), :] = jnp.max(sel, axis=0, keepdims=True)
        return carry

    lax.fori_loop(0, G, body, 0)


def _pool(s0, s1, u_prev, dinv, b, batch2d):
    return pl.pallas_call(
        _pool_body,
        out_shape=[
            jax.ShapeDtypeStruct((G, D), jnp.float32),
            jax.ShapeDtypeStruct((G, D), jnp.float32),
        ],
    )(s0, s1, u_prev, dinv, b, batch2d)


def kernel(x, edge_index, batch, W1, b1, W2, b2, W3, b3):
    row = edge_index[0].astype(jnp.int32)
    col = edge_index[1].astype(jnp.int32)
    ppt = EPT - E // NW
    pad_idx = jnp.arange(NW * ppt, dtype=jnp.int32).reshape(NW, ppt)
    pad_row = (pad_idx * 89) % N
    pad_col = N + (pad_idx % (N_PAD - N))
    row_pad = jnp.concatenate([row.reshape(NW, E // NW), pad_row], axis=1).reshape(-1)
    col_pad = jnp.concatenate([col.reshape(NW, E // NW), pad_col], axis=1).reshape(-1)

    ones_ek = jnp.ones((EK,), jnp.float32)
    zeros_1d = jnp.zeros((ROWS_PER_TILE,), jnp.float32)
    zeros_2d = jnp.zeros((ROWS_PER_TILE, D), jnp.float32)

    deg_parts = _deg_call()(col_pad, ones_ek, zeros_1d)
    deg0 = deg_parts[:N].reshape(N, 1)
    deg1 = deg_parts[N_PAD:N_PAD + N].reshape(N, 1)

    packed_3d = (row_pad | (col_pad << 16)).reshape(NW, G_CHUNKS, GK)

    def agg(u):
        s_parts = _agg_call()(packed_3d, u, zeros_2d)
        return s_parts[:N], s_parts[N_PAD:N_PAD + N]

    u1, dinv = _layer1(x, W1, deg0, deg1)
    s0, s1 = agg(u1)
    u2 = _mid_layer(s0, s1, u1, dinv, b1.reshape(1, D), W2)
    s0, s1 = agg(u2)
    u3 = _mid_layer(s0, s1, u2, dinv, b2.reshape(1, D), W3)
    s0, s1 = agg(u3)
    mean, mx = _pool(s0, s1, u3, dinv, b3.reshape(1, D), batch.reshape(1, N))
    return jnp.concatenate([mean, mx], axis=-1)


# speedup vs baseline: 21.3931x; 1.1089x over previous
import functools

import jax
import jax.numpy as jnp
from jax import lax
from jax.experimental import pallas as pl
from jax.experimental.pallas import tpu as pltpu
from jax.experimental.pallas import tpu_sc as plsc

N = 10000
E = 320000
G = 64
D = 128

NC = 2
NS = 16
NW = NC * NS

N_PAD = 10240
ROWS_PER_TILE = N_PAD // NS
EK = 128
EPT = 10112
E_PAD = EPT * NW
N_CHUNKS = EPT // EK

GK = 64
G_CHUNKS = EPT // GK
NBUF = 2

@functools.cache
def _sc_mesh():
    return plsc.VectorSubcoreMesh(
        core_axis_name="c", subcore_axis_name="s", num_cores=NC, num_subcores=NS)


DK = 128
D_CHUNKS = E_PAD // NS // DK
HALF_N = N_PAD // NC
DPT = HALF_N // NS


def _deg_body(packed_hbm, ones_hbm, zeros_hbm, out_hbm,
              pbuf, ubuf, onesbuf, dvbuf, acc, sem):
    cid = lax.axis_index("c")
    sid = lax.axis_index("s")

    pltpu.sync_copy(packed_hbm.at[2 * sid], pbuf.at[pl.ds(0, D_CHUNKS // 2)])
    pltpu.sync_copy(packed_hbm.at[2 * sid + 1],
                    pbuf.at[pl.ds(D_CHUNKS // 2, D_CHUNKS // 2)])
    pltpu.sync_copy(ones_hbm, onesbuf)
    row0 = pl.multiple_of(sid * ROWS_PER_TILE, 8)
    pltpu.sync_copy(zeros_hbm, acc.at[pl.ds(row0, ROWS_PER_TILE)])

    def unpack(i, carry):
        for j in range(DK // 16):
            ubuf[i, pl.ds(j * 16, 16)] = lax.shift_right_logical(
                pbuf[i, pl.ds(j * 16, 16)], 16)
        return carry

    lax.fori_loop(0, D_CHUNKS, unpack, 0)
    plsc.subcore_barrier()

    pltpu.async_copy(onesbuf, acc.at[ubuf.at[0]], sem, add=True)
    pltpu.async_copy(onesbuf, acc.at[ubuf.at[1]], sem, add=True)

    def body(i, carry):
        pltpu.make_async_copy(onesbuf, acc.at[ubuf.at[i]], sem).wait()
        pltpu.async_copy(onesbuf, acc.at[ubuf.at[i + 2]], sem, add=True)
        return carry

    lax.fori_loop(0, D_CHUNKS - 2, body, 0)
    for i in range(D_CHUNKS - 2, D_CHUNKS):
        pltpu.make_async_copy(onesbuf, acc.at[ubuf.at[i]], sem).wait()
    plsc.subcore_barrier()

    half0 = pl.multiple_of(cid * HALF_N + sid * DPT, 8)
    pltpu.sync_copy(acc.at[pl.ds(half0, DPT)], dvbuf)
    for k in range(DPT // 16):
        x = dvbuf[pl.ds(k * 16, 16)] + 1.0
        i32 = lax.bitcast_convert_type(x, jnp.int32)
        seed = lax.bitcast_convert_type(
            0x5F3759DF - lax.shift_right_logical(i32, 1), jnp.float32)
        y = seed
        for _ in range(3):
            y = y * (1.5 - 0.5 * x * y * y)
        dvbuf[pl.ds(k * 16, 16)] = y
    pltpu.sync_copy(dvbuf, out_hbm.at[pl.ds(half0, DPT)])


@functools.cache
def _deg_call():
    return pl.kernel(
        _deg_body,
        out_type=jax.ShapeDtypeStruct((N_PAD,), jnp.float32),
        mesh=_sc_mesh(),
        scratch_types=[
            pltpu.VMEM((D_CHUNKS, DK), jnp.int32),
            pltpu.VMEM((D_CHUNKS, DK), jnp.int32),
            pltpu.VMEM((DK,), jnp.float32),
            pltpu.VMEM((DPT,), jnp.float32),
            pltpu.VMEM_SHARED((N_PAD,), jnp.float32),
            pltpu.SemaphoreType.DMA,
        ],
    )


def _agg_body(packed_hbm, u_hbm, zeros_hbm, out_hbm,
              pbuf, r0buf, r1buf, c0buf, c1buf, gbuf0, gbuf1, acc, sem):
    cid = lax.axis_index("c")
    sid = lax.axis_index("s")
    wid = sid * NC + cid
    gbufs = [gbuf0, gbuf1]
    rbufs = [r0buf, r1buf]
    cbufs = [c0buf, c1buf]

    pltpu.sync_copy(packed_hbm.at[wid], pbuf)

    row0 = pl.multiple_of(sid * ROWS_PER_TILE, 8)
    pltpu.sync_copy(zeros_hbm, acc.at[pl.ds(row0, ROWS_PER_TILE)])

    def unpack(i, b):
        for j in range(GK // 16):
            v = pbuf[i, pl.ds(j * 16, 16)]
            rbufs[b][pl.ds(j * 16, 16)] = lax.bitwise_and(v, 0xFFFF)
            cbufs[b][pl.ds(j * 16, 16)] = lax.shift_right_logical(v, 16)

    plsc.subcore_barrier()

    for b in range(NBUF):
        unpack(b, b)
        pltpu.async_copy(u_hbm.at[rbufs[b]], gbufs[b], sem)

    def body(i0, carry):
        for b in range(NBUF):
            i = i0 * NBUF + b
            pltpu.make_async_copy(u_hbm.at[rbufs[b]], gbufs[b], sem).wait()
            pltpu.sync_copy(gbufs[b], acc.at[cbufs[b]], add=True)
            unpack(i + NBUF, b)
            pltpu.async_copy(u_hbm.at[rbufs[b]], gbufs[b], sem)
        return carry

    lax.fori_loop(0, G_CHUNKS // NBUF - 1, body, 0)
    for b in range(NBUF):
        pltpu.make_async_copy(u_hbm.at[rbufs[b]], gbufs[b], sem).wait()
        pltpu.sync_copy(gbufs[b], acc.at[cbufs[b]], add=True)
    plsc.subcore_barrier()

    out0 = pl.multiple_of(cid * N_PAD + sid * ROWS_PER_TILE, 8)
    pltpu.sync_copy(acc.at[pl.ds(row0, ROWS_PER_TILE)],
                    out_hbm.at[pl.ds(out0, ROWS_PER_TILE)])


@functools.cache
def _agg_call():
    return pl.kernel(
        _agg_body,
        out_type=jax.ShapeDtypeStruct((NC * N_PAD, D), jnp.float32),
        mesh=_sc_mesh(),
        scratch_types=[
            pltpu.VMEM((G_CHUNKS, GK), jnp.int32),
            pltpu.VMEM((GK,), jnp.int32),
            pltpu.VMEM((GK,), jnp.int32),
            pltpu.VMEM((GK,), jnp.int32),
            pltpu.VMEM((GK,), jnp.int32),
            pltpu.VMEM((GK, D), jnp.float32),
            pltpu.VMEM((GK, D), jnp.float32),
            pltpu.VMEM_SHARED((N_PAD, D), jnp.float32),
            pltpu.SemaphoreType.DMA,
        ],
    )


_BLK = 1000


def _layer1_body(x_ref, w_ref, dinv_ref, u_ref):
    h = jnp.dot(x_ref[...], w_ref[...], preferred_element_type=jnp.float32)
    u_ref[...] = dinv_ref[...] * h


def _layer1(x, W, dinv):
    return pl.pallas_call(
        _layer1_body,
        grid=(N // _BLK,),
        in_specs=[
            pl.BlockSpec((_BLK, D), lambda i: (i, 0)),
            pl.BlockSpec((D, D), lambda i: (0, 0)),
            pl.BlockSpec((_BLK, 1), lambda i: (i, 0)),
        ],
        out_specs=pl.BlockSpec((_BLK, D), lambda i: (i, 0)),
        out_shape=jax.ShapeDtypeStruct((N, D), jnp.float32),
    )(x, W, dinv)


def _mid_body(s_ref, up_ref, dinv_ref, b_ref, w_ref, u_ref):
    dinv = dinv_ref[...]
    s = s_ref[0] + s_ref[1] + up_ref[...]
    y = jax.nn.relu(dinv * s + b_ref[...])
    u_ref[...] = dinv * jnp.dot(y, w_ref[...], preferred_element_type=jnp.float32)


def _mid_layer(s_parts3, u_prev, dinv, b, W):
    return pl.pallas_call(
        _mid_body,
        grid=(N // _BLK,),
        in_specs=[
            pl.BlockSpec((2, _BLK, D), lambda i: (0, i, 0)),
            pl.BlockSpec((_BLK, D), lambda i: (i, 0)),
            pl.BlockSpec((_BLK, 1), lambda i: (i, 0)),
            pl.BlockSpec((1, D), lambda i: (0, 0)),
            pl.BlockSpec((D, D), lambda i: (0, 0)),
        ],
        out_specs=pl.BlockSpec((_BLK, D), lambda i: (i, 0)),
        out_shape=jax.ShapeDtypeStruct((N, D), jnp.float32),
    )(s_parts3, u_prev, dinv, b, W)


def _pool_body(s_ref, up_ref, dinv_ref, b_ref, batch_ref,
               mean_ref, max_ref):
    s = s_ref[0, :N, :] + s_ref[1, :N, :] + up_ref[...]
    y = jax.nn.relu(dinv_ref[...] * s + b_ref[...])
    batch_row = batch_ref[...]
    gids = lax.broadcasted_iota(jnp.int32, (G, N), 0)
    mask = (batch_row == gids).astype(jnp.float32)
    seg_sum = jnp.dot(mask, y, preferred_element_type=jnp.float32)
    counts = jnp.sum(mask, axis=1, keepdims=True)
    mean_ref[...] = seg_sum / jnp.maximum(counts, 1.0)

    batch_col = batch_ref[...].reshape(N, 1)
    neg_inf = jnp.float32(-jnp.inf)

    def body(g, carry):
        sel = jnp.where(batch_col == g, y, neg_inf)
        max_ref[pl.ds(g, 1), :] = jnp.max(sel, axis=0, keepdims=True)
        return carry

    lax.fori_loop(0, G, body, 0)


def _pool(s_parts3, u_prev, dinv, b, batch2d):
    return pl.pallas_call(
        _pool_body,
        out_shape=[
            jax.ShapeDtypeStruct((G, D), jnp.float32),
            jax.ShapeDtypeStruct((G, D), jnp.float32),
        ],
    )(s_parts3, u_prev, dinv, b, batch2d)


def kernel(x, edge_index, batch, W1, b1, W2, b2, W3, b3):
    row = edge_index[0].astype(jnp.int32)
    col = edge_index[1].astype(jnp.int32)
    ppt = EPT - E // NW
    pad_idx = jnp.arange(NW * ppt, dtype=jnp.int32).reshape(NW, ppt)
    pad_row = (pad_idx * 89) % N
    pad_col = N + (pad_idx % (N_PAD - N))
    row_pad = jnp.concatenate([row.reshape(NW, E // NW), pad_row], axis=1).reshape(-1)
    col_pad = jnp.concatenate([col.reshape(NW, E // NW), pad_col], axis=1).reshape(-1)

    ones_dk = jnp.ones((DK,), jnp.float32)
    zeros_1d = jnp.zeros((ROWS_PER_TILE,), jnp.float32)
    zeros_2d = jnp.zeros((ROWS_PER_TILE, D), jnp.float32)

    packed = row_pad | (col_pad << 16)
    packed_3d = packed.reshape(NW, G_CHUNKS, GK)
    packed_deg = packed.reshape(NW, EPT // DK, DK)

    dinv_pad = _deg_call()(packed_deg, ones_dk, zeros_1d)
    dinv = dinv_pad[:N].reshape(N, 1)

    def agg(u):
        s_parts = _agg_call()(packed_3d, u, zeros_2d)
        return s_parts.reshape(NC, N_PAD, D)

    u1 = _layer1(x, W1, dinv)
    sp = agg(u1)
    u2 = _mid_layer(sp, u1, dinv, b1.reshape(1, D), W2)
    sp = agg(u2)
    u3 = _mid_layer(sp, u2, dinv, b2.reshape(1, D), W3)
    sp = agg(u3)
    mean, mx = _pool(sp, u3, dinv, b3.reshape(1, D), batch.reshape(1, N))
    return jnp.concatenate([mean, mx], axis=-1)


# speedup vs baseline: 23.2969x; 1.0890x over previous
import functools

import jax
import jax.numpy as jnp
from jax import lax
from jax.experimental import pallas as pl
from jax.experimental.pallas import tpu as pltpu
from jax.experimental.pallas import tpu_sc as plsc

N = 10000
E = 320000
G = 64
D = 128

NC = 2
NS = 16
NW = NC * NS

N_PAD = 10240
ROWS_PER_TILE = N_PAD // NS
EK = 128
EPT = 10112
E_PAD = EPT * NW
N_CHUNKS = EPT // EK

GK = 64
G_CHUNKS = EPT // GK
NBUF = 2

@functools.cache
def _sc_mesh():
    return plsc.VectorSubcoreMesh(
        core_axis_name="c", subcore_axis_name="s", num_cores=NC, num_subcores=NS)


DK = 128
D_CHUNKS = E_PAD // NS // DK
HALF_N = N_PAD // NC
DPT = HALF_N // NS


def _deg_body(packed_hbm, ones_hbm, zeros_hbm, out_hbm,
              pbuf, ubuf, onesbuf, dvbuf, acc, sem):
    cid = lax.axis_index("c")
    sid = lax.axis_index("s")

    pltpu.sync_copy(packed_hbm.at[2 * sid], pbuf.at[pl.ds(0, D_CHUNKS // 2)])
    pltpu.sync_copy(packed_hbm.at[2 * sid + 1],
                    pbuf.at[pl.ds(D_CHUNKS // 2, D_CHUNKS // 2)])
    pltpu.sync_copy(ones_hbm, onesbuf)
    row0 = pl.multiple_of(sid * ROWS_PER_TILE, 8)
    pltpu.sync_copy(zeros_hbm, acc.at[pl.ds(row0, ROWS_PER_TILE)])

    def unpack(i, carry):
        for j in range(DK // 16):
            ubuf[i, pl.ds(j * 16, 16)] = lax.shift_right_logical(
                pbuf[i, pl.ds(j * 16, 16)], 16)
        return carry

    lax.fori_loop(0, D_CHUNKS, unpack, 0)
    plsc.subcore_barrier()

    pltpu.async_copy(onesbuf, acc.at[ubuf.at[0]], sem, add=True)
    pltpu.async_copy(onesbuf, acc.at[ubuf.at[1]], sem, add=True)

    def body(i, carry):
        pltpu.make_async_copy(onesbuf, acc.at[ubuf.at[i]], sem).wait()
        pltpu.async_copy(onesbuf, acc.at[ubuf.at[i + 2]], sem, add=True)
        return carry

    lax.fori_loop(0, D_CHUNKS - 2, body, 0)
    for i in range(D_CHUNKS - 2, D_CHUNKS):
        pltpu.make_async_copy(onesbuf, acc.at[ubuf.at[i]], sem).wait()
    plsc.subcore_barrier()

    half0 = pl.multiple_of(cid * HALF_N + sid * DPT, 8)
    pltpu.sync_copy(acc.at[pl.ds(half0, DPT)], dvbuf)
    for k in range(DPT // 16):
        x = dvbuf[pl.ds(k * 16, 16)] + 1.0
        i32 = lax.bitcast_convert_type(x, jnp.int32)
        seed = lax.bitcast_convert_type(
            0x5F3759DF - lax.shift_right_logical(i32, 1), jnp.float32)
        y = seed
        for _ in range(3):
            y = y * (1.5 - 0.5 * x * y * y)
        dvbuf[pl.ds(k * 16, 16)] = y
    pltpu.sync_copy(dvbuf, out_hbm.at[pl.ds(half0, DPT)])


@functools.cache
def _deg_call():
    return pl.kernel(
        _deg_body,
        out_type=jax.ShapeDtypeStruct((N_PAD,), jnp.float32),
        mesh=_sc_mesh(),
        scratch_types=[
            pltpu.VMEM((D_CHUNKS, DK), jnp.int32),
            pltpu.VMEM((D_CHUNKS, DK), jnp.int32),
            pltpu.VMEM((DK,), jnp.float32),
            pltpu.VMEM((DPT,), jnp.float32),
            pltpu.VMEM_SHARED((N_PAD,), jnp.float32),
            pltpu.SemaphoreType.DMA,
        ],
    )


def _agg_body(packed_hbm, u_hbm, zeros_hbm, out_hbm,
              pbuf, r0buf, r1buf, c0buf, c1buf, gbuf0, gbuf1, acc, sem):
    cid = lax.axis_index("c")
    sid = lax.axis_index("s")
    wid = sid * NC + cid
    gbufs = [gbuf0, gbuf1]
    rbufs = [r0buf, r1buf]
    cbufs = [c0buf, c1buf]

    pltpu.sync_copy(packed_hbm.at[wid], pbuf)

    row0 = pl.multiple_of(sid * ROWS_PER_TILE, 8)
    pltpu.sync_copy(zeros_hbm, acc.at[pl.ds(row0, ROWS_PER_TILE)])

    def unpack(i, b):
        for j in range(GK // 16):
            v = pbuf[i, pl.ds(j * 16, 16)]
            rbufs[b][pl.ds(j * 16, 16)] = lax.bitwise_and(v, 0xFFFF)
            cbufs[b][pl.ds(j * 16, 16)] = lax.shift_right_logical(v, 16)

    plsc.subcore_barrier()

    for b in range(NBUF):
        unpack(b, b)
        pltpu.async_copy(u_hbm.at[rbufs[b]], gbufs[b], sem)

    def body(i0, carry):
        for b in range(NBUF):
            i = i0 * NBUF + b
            pltpu.make_async_copy(u_hbm.at[rbufs[b]], gbufs[b], sem).wait()
            pltpu.sync_copy(gbufs[b], acc.at[cbufs[b]], add=True)
            unpack(i + NBUF, b)
            pltpu.async_copy(u_hbm.at[rbufs[b]], gbufs[b], sem)
        return carry

    lax.fori_loop(0, G_CHUNKS // NBUF - 1, body, 0)
    for b in range(NBUF):
        pltpu.make_async_copy(u_hbm.at[rbufs[b]], gbufs[b], sem).wait()
        pltpu.sync_copy(gbufs[b], acc.at[cbufs[b]], add=True)
    plsc.subcore_barrier()

    out0 = pl.multiple_of(cid * N_PAD + sid * ROWS_PER_TILE, 8)
    pltpu.sync_copy(acc.at[pl.ds(row0, ROWS_PER_TILE)],
                    out_hbm.at[pl.ds(out0, ROWS_PER_TILE)])


@functools.cache
def _agg_call():
    return pl.kernel(
        _agg_body,
        out_type=jax.ShapeDtypeStruct((NC * N_PAD, D), jnp.float32),
        mesh=_sc_mesh(),
        scratch_types=[
            pltpu.VMEM((G_CHUNKS, GK), jnp.int32),
            pltpu.VMEM((GK,), jnp.int32),
            pltpu.VMEM((GK,), jnp.int32),
            pltpu.VMEM((GK,), jnp.int32),
            pltpu.VMEM((GK,), jnp.int32),
            pltpu.VMEM((GK, D), jnp.float32),
            pltpu.VMEM((GK, D), jnp.float32),
            pltpu.VMEM_SHARED((N_PAD, D), jnp.float32),
            pltpu.SemaphoreType.DMA,
        ],
    )


_BLK = 1000


def _layer1_body(x_ref, w_ref, dinv_ref, u_ref):
    h = jnp.dot(x_ref[...], w_ref[...], preferred_element_type=jnp.float32)
    u_ref[...] = dinv_ref[...] * h


def _layer1(x, W, dinv):
    return pl.pallas_call(
        _layer1_body,
        grid=(N // _BLK,),
        in_specs=[
            pl.BlockSpec((_BLK, D), lambda i: (i, 0)),
            pl.BlockSpec((D, D), lambda i: (0, 0)),
            pl.BlockSpec((_BLK, 1), lambda i: (i, 0)),
        ],
        out_specs=pl.BlockSpec((_BLK, D), lambda i: (i, 0)),
        out_shape=jax.ShapeDtypeStruct((N, D), jnp.float32),
    )(x, W, dinv)


def _mid_body(s_ref, up_ref, dinv_ref, b_ref, w_ref, u_ref):
    dinv = dinv_ref[...]
    s = s_ref[0] + s_ref[1] + up_ref[...]
    y = jax.nn.relu(dinv * s + b_ref[...])
    u_ref[...] = dinv * jnp.dot(y, w_ref[...], preferred_element_type=jnp.float32)


def _mid_layer(s_parts3, u_prev, dinv, b, W):
    return pl.pallas_call(
        _mid_body,
        grid=(N // _BLK,),
        in_specs=[
            pl.BlockSpec((2, _BLK, D), lambda i: (0, i, 0)),
            pl.BlockSpec((_BLK, D), lambda i: (i, 0)),
            pl.BlockSpec((_BLK, 1), lambda i: (i, 0)),
            pl.BlockSpec((1, D), lambda i: (0, 0)),
            pl.BlockSpec((D, D), lambda i: (0, 0)),
        ],
        out_specs=pl.BlockSpec((_BLK, D), lambda i: (i, 0)),
        out_shape=jax.ShapeDtypeStruct((N, D), jnp.float32),
    )(s_parts3, u_prev, dinv, b, W)


def _y3_body(s_ref, up_ref, dinv_ref, b_ref, y_ref):
    s = s_ref[0] + s_ref[1] + up_ref[...]
    y_ref[...] = jax.nn.relu(dinv_ref[...] * s + b_ref[...])


def _y3(s_parts3, u_prev, dinv, b):
    return pl.pallas_call(
        _y3_body,
        grid=(N // _BLK,),
        in_specs=[
            pl.BlockSpec((2, _BLK, D), lambda i: (0, i, 0)),
            pl.BlockSpec((_BLK, D), lambda i: (i, 0)),
            pl.BlockSpec((_BLK, 1), lambda i: (i, 0)),
            pl.BlockSpec((1, D), lambda i: (0, 0)),
        ],
        out_specs=pl.BlockSpec((_BLK, D), lambda i: (i, 0)),
        out_shape=jax.ShapeDtypeStruct((N, D), jnp.float32),
    )(s_parts3, u_prev, dinv, b)


RPT = 320
GPT = G // NS


def _pool_sc_body(y_hbm, batch_hbm, maxo, sumo, cnto,
                  ybuf, bbuf, mtab, stab, ctab, t4a, t4b, c4a, c4b,
                  shm, shs, shc, sem):
    cid = lax.axis_index("c")
    sid = lax.axis_index("s")
    wid = sid * NC + cid
    last = jnp.int32(NW - 1)
    base = pl.multiple_of(jnp.where(wid == last, N - RPT, wid * RPT), 8)
    rstart = jnp.where(wid == last, NW * RPT - N, 0)

    pltpu.sync_copy(y_hbm.at[pl.ds(base, RPT)], ybuf)
    pltpu.sync_copy(batch_hbm.at[pl.ds(base, RPT)], bbuf.at[pl.ds(0, RPT)])

    neg_inf = jnp.full((16,), -jnp.inf, jnp.float32)
    zeros16 = jnp.zeros((16,), jnp.float32)

    def init(g, carry):
        for j in range(D // 16):
            mtab[g, pl.ds(j * 16, 16)] = neg_inf
            stab[g, pl.ds(j * 16, 16)] = zeros16
        ctab[g, pl.ds(0, 16)] = zeros16
        return carry

    lax.fori_loop(0, G, init, 0)

    def row(r, carry):
        g = bbuf[pl.ds(r, 16)][0]
        ctab[g, pl.ds(0, 16)] = ctab[g, pl.ds(0, 16)] + 1.0
        for j in range(D // 16):
            sl = pl.ds(j * 16, 16)
            v = ybuf[r, sl]
            mtab[g, sl] = jnp.maximum(mtab[g, sl], v)
            stab[g, sl] = stab[g, sl] + v
        return carry

    lax.fori_loop(rstart, RPT, row, 0)

    for q in range(NS):
        pltpu.async_copy(mtab.at[pl.ds(q * GPT, GPT)], shm.at[q, sid], sem)
        pltpu.async_copy(stab.at[pl.ds(q * GPT, GPT)], shs.at[q, sid], sem)
        pltpu.async_copy(ctab.at[pl.ds(q * GPT, GPT)], shc.at[q, sid], sem)
    for q in range(NS):
        pltpu.make_async_copy(mtab.at[pl.ds(q * GPT, GPT)], shm.at[q, sid], sem).wait()
        pltpu.make_async_copy(stab.at[pl.ds(q * GPT, GPT)], shs.at[q, sid], sem).wait()
        pltpu.make_async_copy(ctab.at[pl.ds(q * GPT, GPT)], shc.at[q, sid], sem).wait()
    plsc.subcore_barrier()

    def acc_t(t, carry):
        pltpu.sync_copy(shm.at[sid, t], t4a)
        pltpu.sync_copy(shs.at[sid, t], t4b)
        pltpu.sync_copy(shc.at[sid, t], c4a)
        for k in range(GPT):
            for j in range(D // 16):
                sl = pl.ds(j * 16, 16)
                mtab[k, sl] = jnp.maximum(mtab[k, sl], t4a[k, sl])
                stab[k, sl] = stab[k, sl] + t4b[k, sl]
            ctab[k, pl.ds(0, 16)] = ctab[k, pl.ds(0, 16)] + c4a[k, pl.ds(0, 16)]
        return carry

    pltpu.sync_copy(shm.at[sid, 0], t4a)
    pltpu.sync_copy(shs.at[sid, 0], t4b)
    pltpu.sync_copy(shc.at[sid, 0], c4a)
    for k in range(GPT):
        for j in range(D // 16):
            sl = pl.ds(j * 16, 16)
            mtab[k, sl] = t4a[k, sl]
            stab[k, sl] = t4b[k, sl]
        ctab[k, pl.ds(0, 16)] = c4a[k, pl.ds(0, 16)]
    lax.fori_loop(1, NS, acc_t, 0)

    g0 = pl.multiple_of(sid * GPT, 4)
    pltpu.sync_copy(mtab.at[pl.ds(0, GPT)], maxo.at[cid, pl.ds(g0, GPT)])
    pltpu.sync_copy(stab.at[pl.ds(0, GPT)], sumo.at[cid, pl.ds(g0, GPT)])
    pltpu.sync_copy(ctab.at[pl.ds(0, GPT)], cnto.at[cid, pl.ds(g0, GPT)])


@functools.cache
def _pool_sc():
    return pl.kernel(
        _pool_sc_body,
        out_type=[
            jax.ShapeDtypeStruct((NC, G, D), jnp.float32),
            jax.ShapeDtypeStruct((NC, G, D), jnp.float32),
            jax.ShapeDtypeStruct((NC, G, 16), jnp.float32),
        ],
        mesh=_sc_mesh(),
        scratch_types=[
            pltpu.VMEM((RPT, D), jnp.float32),
            pltpu.VMEM((RPT + 16,), jnp.int32),
            pltpu.VMEM((G, D), jnp.float32),
            pltpu.VMEM((G, D), jnp.float32),
            pltpu.VMEM((G, 16), jnp.float32),
            pltpu.VMEM((GPT, D), jnp.float32),
            pltpu.VMEM((GPT, D), jnp.float32),
            pltpu.VMEM((GPT, 16), jnp.float32),
            pltpu.VMEM((GPT, 16), jnp.float32),
            pltpu.VMEM_SHARED((NS, NS, GPT, D), jnp.float32),
            pltpu.VMEM_SHARED((NS, NS, GPT, D), jnp.float32),
            pltpu.VMEM_SHARED((NS, NS, GPT, 16), jnp.float32),
            pltpu.SemaphoreType.DMA,
        ],
    )


def _finish_body(maxt_ref, sumt_ref, cntt_ref, out_ref):
    cnt = cntt_ref[0, :, 0:1] + cntt_ref[1, :, 0:1]
    mean = (sumt_ref[0] + sumt_ref[1]) / jnp.maximum(cnt, 1.0)
    mx = jnp.maximum(maxt_ref[0], maxt_ref[1])
    out_ref[:, :D] = mean
    out_ref[:, D:] = mx


def _finish(maxt, sumt, cntt):
    return pl.pallas_call(
        _finish_body,
        out_shape=jax.ShapeDtypeStruct((G, 2 * D), jnp.float32),
    )(maxt, sumt, cntt)


def kernel(x, edge_index, batch, W1, b1, W2, b2, W3, b3):
    row = edge_index[0].astype(jnp.int32)
    col = edge_index[1].astype(jnp.int32)
    ppt = EPT - E // NW
    pad_idx = jnp.arange(NW * ppt, dtype=jnp.int32).reshape(NW, ppt)
    pad_row = (pad_idx * 89) % N
    pad_col = N + (pad_idx % (N_PAD - N))
    row_pad = jnp.concatenate([row.reshape(NW, E // NW), pad_row], axis=1).reshape(-1)
    col_pad = jnp.concatenate([col.reshape(NW, E // NW), pad_col], axis=1).reshape(-1)

    ones_dk = jnp.ones((DK,), jnp.float32)
    zeros_1d = jnp.zeros((ROWS_PER_TILE,), jnp.float32)
    zeros_2d = jnp.zeros((ROWS_PER_TILE, D), jnp.float32)

    packed = row_pad | (col_pad << 16)
    packed_3d = packed.reshape(NW, G_CHUNKS, GK)
    packed_deg = packed.reshape(NW, EPT // DK, DK)

    dinv_pad = _deg_call()(packed_deg, ones_dk, zeros_1d)
    dinv = dinv_pad[:N].reshape(N, 1)

    def agg(u):
        s_parts = _agg_call()(packed_3d, u, zeros_2d)
        return s_parts.reshape(NC, N_PAD, D)

    u1 = _layer1(x, W1, dinv)
    sp = agg(u1)
    u2 = _mid_layer(sp, u1, dinv, b1.reshape(1, D), W2)
    sp = agg(u2)
    u3 = _mid_layer(sp, u2, dinv, b2.reshape(1, D), W3)
    sp = agg(u3)
    y3 = _y3(sp, u3, dinv, b3.reshape(1, D))
    maxt, sumt, cntt = _pool_sc()(y3, batch)
    return _finish(maxt, sumt, cntt)


# speedup vs baseline: 26.1754x; 1.1236x over previous
import functools

import jax
import jax.numpy as jnp
from jax import lax
from jax.experimental import pallas as pl
from jax.experimental.pallas import tpu as pltpu
from jax.experimental.pallas import tpu_sc as plsc

N = 10000
E = 320000
G = 64
D = 128

NC = 2
NS = 16
NW = NC * NS

NP_DEG = 10240
RPT_DEG = NP_DEG // NS
NP_AGG = 10112
RPT_AGG = NP_AGG // NS
EPT = 10080
E_PAD = EPT * NW

GK = 48
G_CHUNKS = EPT // GK
NBUF = 3

@functools.cache
def _sc_mesh():
    return plsc.VectorSubcoreMesh(
        core_axis_name="c", subcore_axis_name="s", num_cores=NC, num_subcores=NS)


DK = 96
D_CHUNKS = E_PAD // NS // DK
HALF_N = NP_DEG // NC
DPT = HALF_N // NS


def _deg_body(packed_hbm, ones_hbm, zeros_hbm, out_hbm,
              pbuf, ubuf, onesbuf, dvbuf, acc, sem):
    cid = lax.axis_index("c")
    sid = lax.axis_index("s")

    pltpu.sync_copy(packed_hbm.at[sid], pbuf)
    pltpu.sync_copy(ones_hbm, onesbuf)
    row0 = pl.multiple_of(sid * RPT_DEG, 8)
    pltpu.sync_copy(zeros_hbm, acc.at[pl.ds(row0, RPT_DEG)])

    def unpack(i, carry):
        for j in range(DK // 16):
            ubuf[i, pl.ds(j * 16, 16)] = lax.shift_right_logical(
                pbuf[i, pl.ds(j * 16, 16)], 16)
        return carry

    lax.fori_loop(0, D_CHUNKS, unpack, 0)
    plsc.subcore_barrier()

    pltpu.async_copy(onesbuf, acc.at[ubuf.at[0]], sem, add=True)
    pltpu.async_copy(onesbuf, acc.at[ubuf.at[1]], sem, add=True)

    def body(i, carry):
        pltpu.make_async_copy(onesbuf, acc.at[ubuf.at[i]], sem).wait()
        pltpu.async_copy(onesbuf, acc.at[ubuf.at[i + 2]], sem, add=True)
        return carry

    lax.fori_loop(0, D_CHUNKS - 2, body, 0)
    for i in range(D_CHUNKS - 2, D_CHUNKS):
        pltpu.make_async_copy(onesbuf, acc.at[ubuf.at[i]], sem).wait()
    plsc.subcore_barrier()

    half0 = pl.multiple_of(cid * HALF_N + sid * DPT, 8)
    pltpu.sync_copy(acc.at[pl.ds(half0, DPT)], dvbuf)
    for k in range(DPT // 16):
        x = dvbuf[pl.ds(k * 16, 16)] + 1.0
        i32 = lax.bitcast_convert_type(x, jnp.int32)
        seed = lax.bitcast_convert_type(
            0x5F3759DF - lax.shift_right_logical(i32, 1), jnp.float32)
        y = seed
        for _ in range(3):
            y = y * (1.5 - 0.5 * x * y * y)
        dvbuf[pl.ds(k * 16, 16)] = y
    pltpu.sync_copy(dvbuf, out_hbm.at[pl.ds(half0, DPT)])


@functools.cache
def _deg_call():
    return pl.kernel(
        _deg_body,
        out_type=jax.ShapeDtypeStruct((NP_DEG,), jnp.float32),
        mesh=_sc_mesh(),
        scratch_types=[
            pltpu.VMEM((D_CHUNKS, DK), jnp.int32),
            pltpu.VMEM((D_CHUNKS, DK), jnp.int32),
            pltpu.VMEM((DK,), jnp.float32),
            pltpu.VMEM((DPT,), jnp.float32),
            pltpu.VMEM_SHARED((NP_DEG,), jnp.float32),
            pltpu.SemaphoreType.DMA,
        ],
    )


def _agg_body(packed_hbm, u_hbm, zeros_hbm, out_hbm,
              pbuf, r0buf, r1buf, r2buf, c0buf, c1buf, c2buf,
              gbuf0, gbuf1, gbuf2, acc, sem):
    cid = lax.axis_index("c")
    sid = lax.axis_index("s")
    wid = sid * NC + cid
    gbufs = [gbuf0, gbuf1, gbuf2]
    rbufs = [r0buf, r1buf, r2buf]
    cbufs = [c0buf, c1buf, c2buf]

    pltpu.sync_copy(packed_hbm.at[wid], pbuf)

    row0 = pl.multiple_of(sid * RPT_AGG, 8)
    pltpu.sync_copy(zeros_hbm, acc.at[pl.ds(row0, RPT_AGG)])

    def unpack(i, b):
        for j in range(GK // 16):
            v = pbuf[i, pl.ds(j * 16, 16)]
            rbufs[b][pl.ds(j * 16, 16)] = lax.bitwise_and(v, 0xFFFF)
            cbufs[b][pl.ds(j * 16, 16)] = lax.shift_right_logical(v, 16)

    plsc.subcore_barrier()

    for b in range(NBUF):
        unpack(b, b)
        pltpu.async_copy(u_hbm.at[rbufs[b]], gbufs[b], sem)

    def body(i0, carry):
        for b in range(NBUF):
            i = i0 * NBUF + b
            pltpu.make_async_copy(u_hbm.at[rbufs[b]], gbufs[b], sem).wait()
            pltpu.sync_copy(gbufs[b], acc.at[cbufs[b]], add=True)
            unpack(i + NBUF, b)
            pltpu.async_copy(u_hbm.at[rbufs[b]], gbufs[b], sem)
        return carry

    lax.fori_loop(0, G_CHUNKS // NBUF - 1, body, 0)
    for b in range(NBUF):
        pltpu.make_async_copy(u_hbm.at[rbufs[b]], gbufs[b], sem).wait()
        pltpu.sync_copy(gbufs[b], acc.at[cbufs[b]], add=True)
    plsc.subcore_barrier()

    out0 = pl.multiple_of(cid * NP_AGG + sid * RPT_AGG, 8)
    pltpu.sync_copy(acc.at[pl.ds(row0, RPT_AGG)],
                    out_hbm.at[pl.ds(out0, RPT_AGG)])


@functools.cache
def _agg_call():
    return pl.kernel(
        _agg_body,
        out_type=jax.ShapeDtypeStruct((NC * NP_AGG, D), jnp.float32),
        mesh=_sc_mesh(),
        scratch_types=[
            pltpu.VMEM((G_CHUNKS, GK), jnp.int32),
            pltpu.VMEM((GK,), jnp.int32),
            pltpu.VMEM((GK,), jnp.int32),
            pltpu.VMEM((GK,), jnp.int32),
            pltpu.VMEM((GK,), jnp.int32),
            pltpu.VMEM((GK,), jnp.int32),
            pltpu.VMEM((GK,), jnp.int32),
            pltpu.VMEM((GK, D), jnp.float32),
            pltpu.VMEM((GK, D), jnp.float32),
            pltpu.VMEM((GK, D), jnp.float32),
            pltpu.VMEM_SHARED((NP_AGG, D), jnp.float32),
            pltpu.SemaphoreType.DMA,
        ],
    )


_BLK = 1000


def _layer1_body(x_ref, w_ref, dinv_ref, u_ref):
    h = jnp.dot(x_ref[...], w_ref[...], preferred_element_type=jnp.float32)
    u_ref[...] = dinv_ref[...] * h


def _layer1(x, W, dinv):
    return pl.pallas_call(
        _layer1_body,
        grid=(N // _BLK,),
        in_specs=[
            pl.BlockSpec((_BLK, D), lambda i: (i, 0)),
            pl.BlockSpec((D, D), lambda i: (0, 0)),
            pl.BlockSpec((_BLK, 1), lambda i: (i, 0)),
        ],
        out_specs=pl.BlockSpec((_BLK, D), lambda i: (i, 0)),
        out_shape=jax.ShapeDtypeStruct((N, D), jnp.float32),
    )(x, W, dinv)


def _mid_body(s_ref, up_ref, dinv_ref, b_ref, w_ref, u_ref):
    dinv = dinv_ref[...]
    s = s_ref[0] + s_ref[1] + up_ref[...]
    y = jax.nn.relu(dinv * s + b_ref[...])
    u_ref[...] = dinv * jnp.dot(y, w_ref[...], preferred_element_type=jnp.float32)


def _mid_layer(s_parts3, u_prev, dinv, b, W):
    return pl.pallas_call(
        _mid_body,
        grid=(N // _BLK,),
        in_specs=[
            pl.BlockSpec((2, _BLK, D), lambda i: (0, i, 0)),
            pl.BlockSpec((_BLK, D), lambda i: (i, 0)),
            pl.BlockSpec((_BLK, 1), lambda i: (i, 0)),
            pl.BlockSpec((1, D), lambda i: (0, 0)),
            pl.BlockSpec((D, D), lambda i: (0, 0)),
        ],
        out_specs=pl.BlockSpec((_BLK, D), lambda i: (i, 0)),
        out_shape=jax.ShapeDtypeStruct((N, D), jnp.float32),
    )(s_parts3, u_prev, dinv, b, W)


def _y3_body(s_ref, up_ref, dinv_ref, b_ref, y_ref):
    s = s_ref[0] + s_ref[1] + up_ref[...]
    y_ref[...] = jax.nn.relu(dinv_ref[...] * s + b_ref[...])


def _y3(s_parts3, u_prev, dinv, b):
    return pl.pallas_call(
        _y3_body,
        grid=(N // _BLK,),
        in_specs=[
            pl.BlockSpec((2, _BLK, D), lambda i: (0, i, 0)),
            pl.BlockSpec((_BLK, D), lambda i: (i, 0)),
            pl.BlockSpec((_BLK, 1), lambda i: (i, 0)),
            pl.BlockSpec((1, D), lambda i: (0, 0)),
        ],
        out_specs=pl.BlockSpec((_BLK, D), lambda i: (i, 0)),
        out_shape=jax.ShapeDtypeStruct((N, D), jnp.float32),
    )(s_parts3, u_prev, dinv, b)


RPT = 320
GPT = G // NS


def _pool_sc_body(y_hbm, batch_hbm, maxo, sumo, cnto,
                  ybuf, bbuf, mtab, stab, ctab, t4a, t4b, c4a, c4b,
                  shm, shs, shc, sem):
    cid = lax.axis_index("c")
    sid = lax.axis_index("s")
    wid = sid * NC + cid
    last = jnp.int32(NW - 1)
    base = pl.multiple_of(jnp.where(wid == last, N - RPT, wid * RPT), 8)
    rstart = jnp.where(wid == last, NW * RPT - N, 0)

    pltpu.sync_copy(y_hbm.at[pl.ds(base, RPT)], ybuf)
    pltpu.sync_copy(batch_hbm.at[pl.ds(base, RPT)], bbuf.at[pl.ds(0, RPT)])

    neg_inf = jnp.full((16,), -jnp.inf, jnp.float32)
    zeros16 = jnp.zeros((16,), jnp.float32)

    def init(g, carry):
        for j in range(D // 16):
            mtab[g, pl.ds(j * 16, 16)] = neg_inf
            stab[g, pl.ds(j * 16, 16)] = zeros16
        ctab[g, pl.ds(0, 16)] = zeros16
        return carry

    lax.fori_loop(0, G, init, 0)

    def row(r, carry):
        g = bbuf[pl.ds(r, 16)][0]
        ctab[g, pl.ds(0, 16)] = ctab[g, pl.ds(0, 16)] + 1.0
        for j in range(D // 16):
            sl = pl.ds(j * 16, 16)
            v = ybuf[r, sl]
            mtab[g, sl] = jnp.maximum(mtab[g, sl], v)
            stab[g, sl] = stab[g, sl] + v
        return carry

    lax.fori_loop(rstart, RPT, row, 0)

    for q in range(NS):
        pltpu.async_copy(mtab.at[pl.ds(q * GPT, GPT)], shm.at[q, sid], sem)
        pltpu.async_copy(stab.at[pl.ds(q * GPT, GPT)], shs.at[q, sid], sem)
        pltpu.async_copy(ctab.at[pl.ds(q * GPT, GPT)], shc.at[q, sid], sem)
    for q in range(NS):
        pltpu.make_async_copy(mtab.at[pl.ds(q * GPT, GPT)], shm.at[q, sid], sem).wait()
        pltpu.make_async_copy(stab.at[pl.ds(q * GPT, GPT)], shs.at[q, sid], sem).wait()
        pltpu.make_async_copy(ctab.at[pl.ds(q * GPT, GPT)], shc.at[q, sid], sem).wait()
    plsc.subcore_barrier()

    def acc_t(t, carry):
        pltpu.sync_copy(shm.at[sid, t], t4a)
        pltpu.sync_copy(shs.at[sid, t], t4b)
        pltpu.sync_copy(shc.at[sid, t], c4a)
        for k in range(GPT):
            for j in range(D // 16):
                sl = pl.ds(j * 16, 16)
                mtab[k, sl] = jnp.maximum(mtab[k, sl], t4a[k, sl])
                stab[k, sl] = stab[k, sl] + t4b[k, sl]
            ctab[k, pl.ds(0, 16)] = ctab[k, pl.ds(0, 16)] + c4a[k, pl.ds(0, 16)]
        return carry

    pltpu.sync_copy(shm.at[sid, 0], t4a)
    pltpu.sync_copy(shs.at[sid, 0], t4b)
    pltpu.sync_copy(shc.at[sid, 0], c4a)
    for k in range(GPT):
        for j in range(D // 16):
            sl = pl.ds(j * 16, 16)
            mtab[k, sl] = t4a[k, sl]
            stab[k, sl] = t4b[k, sl]
        ctab[k, pl.ds(0, 16)] = c4a[k, pl.ds(0, 16)]
    lax.fori_loop(1, NS, acc_t, 0)

    g0 = pl.multiple_of(sid * GPT, 4)
    pltpu.sync_copy(mtab.at[pl.ds(0, GPT)], maxo.at[cid, pl.ds(g0, GPT)])
    pltpu.sync_copy(stab.at[pl.ds(0, GPT)], sumo.at[cid, pl.ds(g0, GPT)])
    pltpu.sync_copy(ctab.at[pl.ds(0, GPT)], cnto.at[cid, pl.ds(g0, GPT)])


@functools.cache
def _pool_sc():
    return pl.kernel(
        _pool_sc_body,
        out_type=[
            jax.ShapeDtypeStruct((NC, G, D), jnp.float32),
            jax.ShapeDtypeStruct((NC, G, D), jnp.float32),
            jax.ShapeDtypeStruct((NC, G, 16), jnp.float32),
        ],
        mesh=_sc_mesh(),
        scratch_types=[
            pltpu.VMEM((RPT, D), jnp.float32),
            pltpu.VMEM((RPT + 16,), jnp.int32),
            pltpu.VMEM((G, D), jnp.float32),
            pltpu.VMEM((G, D), jnp.float32),
            pltpu.VMEM((G, 16), jnp.float32),
            pltpu.VMEM((GPT, D), jnp.float32),
            pltpu.VMEM((GPT, D), jnp.float32),
            pltpu.VMEM((GPT, 16), jnp.float32),
            pltpu.VMEM((GPT, 16), jnp.float32),
            pltpu.VMEM_SHARED((NS, NS, GPT, D), jnp.float32),
            pltpu.VMEM_SHARED((NS, NS, GPT, D), jnp.float32),
            pltpu.VMEM_SHARED((NS, NS, GPT, 16), jnp.float32),
            pltpu.SemaphoreType.DMA,
        ],
    )


def _finish_body(maxt_ref, sumt_ref, cntt_ref, out_ref):
    cnt = cntt_ref[0, :, 0:1] + cntt_ref[1, :, 0:1]
    mean = (sumt_ref[0] + sumt_ref[1]) / jnp.maximum(cnt, 1.0)
    mx = jnp.maximum(maxt_ref[0], maxt_ref[1])
    out_ref[:, :D] = mean
    out_ref[:, D:] = mx


def _finish(maxt, sumt, cntt):
    return pl.pallas_call(
        _finish_body,
        out_shape=jax.ShapeDtypeStruct((G, 2 * D), jnp.float32),
    )(maxt, sumt, cntt)


def kernel(x, edge_index, batch, W1, b1, W2, b2, W3, b3):
    row = edge_index[0].astype(jnp.int32)
    col = edge_index[1].astype(jnp.int32)
    ppt = EPT - E // NW
    pad_idx = jnp.arange(NW * ppt, dtype=jnp.int32).reshape(NW, ppt)
    pad_row = (pad_idx * 89) % N
    pad_col = N + (pad_idx % (NP_AGG - N))
    row_pad = jnp.concatenate([row.reshape(NW, E // NW), pad_row], axis=1).reshape(-1)
    col_pad = jnp.concatenate([col.reshape(NW, E // NW), pad_col], axis=1).reshape(-1)

    ones_dk = jnp.ones((DK,), jnp.float32)
    zeros_1d = jnp.zeros((RPT_DEG,), jnp.float32)
    zeros_2d = jnp.zeros((RPT_AGG, D), jnp.float32)

    packed = row_pad | (col_pad << 16)
    packed_3d = packed.reshape(NW, G_CHUNKS, GK)
    packed_deg = packed.reshape(NS, D_CHUNKS, DK)

    dinv_pad = _deg_call()(packed_deg, ones_dk, zeros_1d)
    dinv = dinv_pad[:N].reshape(N, 1)

    def agg(u):
        s_parts = _agg_call()(packed_3d, u, zeros_2d)
        return s_parts.reshape(NC, NP_AGG, D)

    u1 = _layer1(x, W1, dinv)
    sp = agg(u1)
    u2 = _mid_layer(sp, u1, dinv, b1.reshape(1, D), W2)
    sp = agg(u2)
    u3 = _mid_layer(sp, u2, dinv, b2.reshape(1, D), W3)
    sp = agg(u3)
    y3 = _y3(sp, u3, dinv, b3.reshape(1, D))
    maxt, sumt, cntt = _pool_sc()(y3, batch)
    return _finish(maxt, sumt, cntt)
